# Initial kernel scaffold; baseline (speedup 1.0000x reference)
import jax
import jax.numpy as jnp
from jax.experimental import pallas as pl


def kernel(x, router_w, router_bias, gate_w, up_w, down_w, sg_w, su_w, sd_w):
    raise NotImplementedError("write your pallas kernel here")


# speedup vs baseline: 1.5852x; 1.5852x over previous
import functools

import jax
import jax.numpy as jnp
from jax.experimental import pallas as pl
from jax.experimental.pallas import tpu as pltpu

B, T, D = 2, 2048, 1024
E, TOPK, H = 8, 2, 512
SH = H * TOPK
N = B * T
NE = E + TOPK
TM = 512


def _fused_moe_body(rw_ref, rb_ref, x_ref, gw_ref, uw_ref, dw_ref, out_ref):
    e = pl.program_id(1)
    x = x_ref[...]

    logits = jnp.dot(x, rw_ref[...].T, preferred_element_type=jnp.float32)
    logits = logits + rb_ref[...]
    scores = jax.nn.softmax(logits, axis=-1)
    s1 = jnp.max(scores, axis=-1, keepdims=True)
    i1 = jnp.argmax(scores, axis=-1).reshape(TM, 1)
    cols = jax.lax.broadcasted_iota(jnp.int32, (TM, E), 1)
    masked = jnp.where(cols == i1, -jnp.inf, scores)
    s2 = jnp.max(masked, axis=-1, keepdims=True)
    i2 = jnp.argmax(masked, axis=-1).reshape(TM, 1)
    denom = s1 + s2
    w1 = s1 / denom
    w2 = s2 / denom
    w = jnp.where(i1 == e, w1, 0.0) + jnp.where(i2 == e, w2, 0.0)
    w = jnp.where(e >= E, 1.0, w)

    g = jnp.dot(x, gw_ref[0].T, preferred_element_type=jnp.float32)
    u = jnp.dot(x, uw_ref[0].T, preferred_element_type=jnp.float32)
    h = (g * jax.nn.sigmoid(g)) * u
    p = jnp.dot(h, dw_ref[0].T, preferred_element_type=jnp.float32)
    contrib = w * p

    @pl.when(e == 0)
    def _():
        out_ref[...] = contrib

    @pl.when(e != 0)
    def _():
        out_ref[...] += contrib


@jax.jit
def kernel(x, router_w, router_bias, gate_w, up_w, down_w, sg_w, su_w, sd_w):
    flat = x.reshape(N, D)
    gw = jnp.concatenate([gate_w, sg_w.reshape(TOPK, H, D)], axis=0)
    uw = jnp.concatenate([up_w, su_w.reshape(TOPK, H, D)], axis=0)
    sd_split = jnp.stack([sd_w[:, :H], sd_w[:, H:]], axis=0)
    dw = jnp.concatenate([down_w, sd_split], axis=0)
    rb = router_bias.reshape(1, E)

    grid = (N // TM, NE)
    out = pl.pallas_call(
        _fused_moe_body,
        grid=grid,
        in_specs=[
            pl.BlockSpec((E, D), lambda t, e: (0, 0)),
            pl.BlockSpec((1, E), lambda t, e: (0, 0)),
            pl.BlockSpec((TM, D), lambda t, e: (t, 0)),
            pl.BlockSpec((1, H, D), lambda t, e: (e, 0, 0)),
            pl.BlockSpec((1, H, D), lambda t, e: (e, 0, 0)),
            pl.BlockSpec((1, D, H), lambda t, e: (e, 0, 0)),
        ],
        out_specs=pl.BlockSpec((TM, D), lambda t, e: (t, 0)),
        out_shape=jax.ShapeDtypeStruct((N, D), jnp.float32),
        compiler_params=pltpu.CompilerParams(
            dimension_semantics=("parallel", "arbitrary"),
        ),
    )(router_w, rb, flat, gw, uw, dw)
    return out.reshape(B, T, D)


# speedup vs baseline: 1.7825x; 1.1245x over previous
import functools

import jax
import jax.numpy as jnp
from jax.experimental import pallas as pl
from jax.experimental.pallas import tpu as pltpu

B, T, D = 2, 2048, 1024
E, TOPK, H = 8, 2, 512
SH = H * TOPK
N = B * T
NE = E + TOPK
TM = 2048


def _fused_moe_body(rw_ref, rb_ref, x_ref, gw_ref, uw_ref, dw_ref, out_ref):
    e = pl.program_id(1)
    x = x_ref[...]

    logits = jnp.dot(x, rw_ref[...].T, preferred_element_type=jnp.float32)
    logits = logits + rb_ref[...]
    scores = jax.nn.softmax(logits, axis=-1)
    s1 = jnp.max(scores, axis=-1, keepdims=True)
    i1 = jnp.argmax(scores, axis=-1).reshape(TM, 1)
    cols = jax.lax.broadcasted_iota(jnp.int32, (TM, E), 1)
    masked = jnp.where(cols == i1, -jnp.inf, scores)
    s2 = jnp.max(masked, axis=-1, keepdims=True)
    i2 = jnp.argmax(masked, axis=-1).reshape(TM, 1)
    denom = s1 + s2
    w1 = s1 / denom
    w2 = s2 / denom
    w = jnp.where(i1 == e, w1, 0.0) + jnp.where(i2 == e, w2, 0.0)
    w = jnp.where(e >= E, 1.0, w)

    g = jnp.dot(x, gw_ref[0].T, preferred_element_type=jnp.float32)
    u = jnp.dot(x, uw_ref[0].T, preferred_element_type=jnp.float32)
    h = (g * jax.nn.sigmoid(g)) * u
    p = jnp.dot(h, dw_ref[0].T, preferred_element_type=jnp.float32)
    contrib = w * p

    @pl.when(e == 0)
    def _():
        out_ref[...] = contrib

    @pl.when(e != 0)
    def _():
        out_ref[...] += contrib


@jax.jit
def kernel(x, router_w, router_bias, gate_w, up_w, down_w, sg_w, su_w, sd_w):
    flat = x.reshape(N, D)
    gw = jnp.concatenate([gate_w, sg_w.reshape(TOPK, H, D)], axis=0)
    uw = jnp.concatenate([up_w, su_w.reshape(TOPK, H, D)], axis=0)
    sd_split = jnp.stack([sd_w[:, :H], sd_w[:, H:]], axis=0)
    dw = jnp.concatenate([down_w, sd_split], axis=0)
    rb = router_bias.reshape(1, E)

    grid = (N // TM, NE)
    out = pl.pallas_call(
        _fused_moe_body,
        grid=grid,
        in_specs=[
            pl.BlockSpec((E, D), lambda t, e: (0, 0)),
            pl.BlockSpec((1, E), lambda t, e: (0, 0)),
            pl.BlockSpec((TM, D), lambda t, e: (t, 0)),
            pl.BlockSpec((1, H, D), lambda t, e: (e, 0, 0)),
            pl.BlockSpec((1, H, D), lambda t, e: (e, 0, 0)),
            pl.BlockSpec((1, D, H), lambda t, e: (e, 0, 0)),
        ],
        out_specs=pl.BlockSpec((TM, D), lambda t, e: (t, 0)),
        out_shape=jax.ShapeDtypeStruct((N, D), jnp.float32),
        compiler_params=pltpu.CompilerParams(
            dimension_semantics=("parallel", "arbitrary"),
        ),
    )(router_w, rb, flat, gw, uw, dw)
    return out.reshape(B, T, D)
